# baseline (speedup 1.0000x reference)
import functools

import jax
import jax.numpy as jnp
from jax.experimental import pallas as pl
from jax.experimental.pallas import tpu as pltpu


def _conv3x3_kernel(x_ref, w_ref, b_ref, o_ref, slab, col, accs, *,
                    H, W, Wp, N, Nc, L, c_in, bt, tap_offsets):
    ph = pw = 1
    prefix = ph * Wp + pw
    data_end = (H - 1 + ph) * Wp + pw + W
    zrow = jnp.zeros((c_in, prefix), jnp.bfloat16)
    zgap = jnp.zeros((c_in, 2 * pw), jnp.bfloat16)
    for bl in range(bt):
        slab[bl, :, :prefix] = zrow
        if L > data_end:
            slab[bl, :, data_end:] = jnp.zeros((c_in, L - data_end), jnp.bfloat16)
        for h in range(H - 1):
            g0 = (h + ph) * Wp + pw + W
            slab[bl, :, g0:g0 + 2 * pw] = zgap
        for h in range(H):
            base = (h + ph) * Wp + pw
            slab[bl, :, base:base + W] = x_ref[bl, :, h, :].astype(jnp.bfloat16)
    for t, s in enumerate(tap_offsets):
        r0 = t * c_in
        for bl in range(bt):
            col[r0:r0 + c_in, bl * Nc:bl * Nc + N] = slab[bl, :, s:s + N]
    accs[...] = jnp.dot(w_ref[...], col[...],
                        preferred_element_type=jnp.float32) + b_ref[...]
    for bl in range(bt):
        for h in range(H):
            o_ref[bl, :, h, :] = accs[:, bl * Nc + h * Wp:bl * Nc + h * Wp + W]


def kernel(x, weight, bias):
    B, C_in, H, W = x.shape
    C_out, _, KH, KW = weight.shape
    assert (KH, KW) == (3, 3)
    ph = pw = 1
    Hp, Wp = H + 2 * ph, W + 2 * pw
    N = H * Wp
    Nc = -(-N // 128) * 128
    L = Hp * Wp + (KW - 1)
    tap_offsets = tuple(kh * Wp + kw for kh in range(KH) for kw in range(KW))

    bt = 1
    for d in (2, 4):
        if B % d == 0:
            bt = d
            break
    grid = (B // bt,)

    w2 = jnp.transpose(weight, (0, 2, 3, 1)).reshape(C_out, KH * KW * C_in)
    w2 = w2.astype(jnp.bfloat16)
    b2 = bias.astype(jnp.float32).reshape(C_out, 1)

    body = functools.partial(
        _conv3x3_kernel, H=H, W=W, Wp=Wp, N=N, Nc=Nc, L=L,
        c_in=C_in, bt=bt, tap_offsets=tap_offsets)

    out = pl.pallas_call(
        body,
        out_shape=jax.ShapeDtypeStruct((B, C_out, H, W), jnp.float32),
        grid=grid,
        in_specs=[
            pl.BlockSpec((bt, C_in, H, W), lambda g: (g, 0, 0, 0)),
            pl.BlockSpec((C_out, KH * KW * C_in), lambda g: (0, 0)),
            pl.BlockSpec((C_out, 1), lambda g: (0, 0)),
        ],
        out_specs=pl.BlockSpec((bt, C_out, H, W), lambda g: (g, 0, 0, 0)),
        scratch_shapes=[
            pltpu.VMEM((bt, C_in, L), jnp.bfloat16),
            pltpu.VMEM((KH * KW * C_in, bt * Nc), jnp.bfloat16),
            pltpu.VMEM((C_out, bt * Nc), jnp.float32),
        ],
        compiler_params=pltpu.CompilerParams(
            dimension_semantics=("parallel",)),
    )(x, w2, b2)
    return out


# speedup vs baseline: 2.5453x; 2.5453x over previous
import functools

import jax
import jax.numpy as jnp
from jax.experimental import pallas as pl
from jax.experimental.pallas import tpu as pltpu


def _conv3x3_kernel(x_ref, w_ref, b_ref, o_ref, slab, col, accs, *,
                    H, W, Wp, N, Nc, L, c_in, bt, tap_offsets):
    ph = pw = 1
    prefix = ph * Wp + pw
    data_end = (H - 1 + ph) * Wp + pw + W
    zrow = jnp.zeros((c_in, prefix), jnp.bfloat16)
    zgap = jnp.zeros((c_in, 2 * pw), jnp.bfloat16)
    for bl in range(bt):
        slab[bl, :, :prefix] = zrow
        if L > data_end:
            slab[bl, :, data_end:] = jnp.zeros((c_in, L - data_end), jnp.bfloat16)
        for h in range(H - 1):
            g0 = (h + ph) * Wp + pw + W
            slab[bl, :, g0:g0 + 2 * pw] = zgap
        for h in range(H):
            base = (h + ph) * Wp + pw
            slab[bl, :, base:base + W] = x_ref[bl, :, h * W:(h + 1) * W].astype(jnp.bfloat16)
    for bl in range(bt):
        for t, s in enumerate(tap_offsets):
            col[bl, t * c_in:(t + 1) * c_in, :N] = slab[bl, :, s:s + N]
        accs[bl] = jnp.dot(w_ref[...], col[bl],
                           preferred_element_type=jnp.float32) + b_ref[...]
    for bl in range(bt):
        for h in range(H):
            o_ref[bl, :, h * W:(h + 1) * W] = accs[bl, :, h * Wp:h * Wp + W]


def kernel(x, weight, bias):
    B, C_in, H, W = x.shape
    C_out, _, KH, KW = weight.shape
    assert (KH, KW) == (3, 3)
    ph = pw = 1
    Hp, Wp = H + 2 * ph, W + 2 * pw
    N = H * Wp
    Nc = -(-N // 128) * 128
    L = Hp * Wp + (KW - 1)
    tap_offsets = tuple(kh * Wp + kw for kh in range(KH) for kw in range(KW))

    bt = 1
    for d in (2, 4):
        if B % d == 0:
            bt = d
            break
    grid = (B // bt,)

    x2 = x.reshape(B, C_in, H * W)
    w2 = jnp.transpose(weight, (0, 2, 3, 1)).reshape(C_out, KH * KW * C_in)
    w2 = w2.astype(jnp.bfloat16)
    b2 = bias.astype(jnp.float32).reshape(C_out, 1)

    body = functools.partial(
        _conv3x3_kernel, H=H, W=W, Wp=Wp, N=N, Nc=Nc, L=L,
        c_in=C_in, bt=bt, tap_offsets=tap_offsets)

    out = pl.pallas_call(
        body,
        out_shape=jax.ShapeDtypeStruct((B, C_out, H * W), jnp.float32),
        grid=grid,
        in_specs=[
            pl.BlockSpec((bt, C_in, H * W), lambda g: (g, 0, 0)),
            pl.BlockSpec((C_out, KH * KW * C_in), lambda g: (0, 0)),
            pl.BlockSpec((C_out, 1), lambda g: (0, 0)),
        ],
        out_specs=pl.BlockSpec((bt, C_out, H * W), lambda g: (g, 0, 0)),
        scratch_shapes=[
            pltpu.VMEM((bt, C_in, L), jnp.bfloat16),
            pltpu.VMEM((bt, KH * KW * C_in, Nc), jnp.bfloat16),
            pltpu.VMEM((bt, C_out, Nc), jnp.float32),
        ],
        compiler_params=pltpu.CompilerParams(
            dimension_semantics=("parallel",)),
    )(x2, w2, b2)
    return out.reshape(B, C_out, H, W)
